# Initial kernel scaffold; baseline (speedup 1.0000x reference)
import jax
import jax.numpy as jnp
from jax.experimental import pallas as pl


def kernel(h, edge_index, edge_w, norm, edge_feat, proj_w, proj_b, proj_ln_g, proj_ln_b, mp_w, mp_b, mp_ln_g, mp_ln_b, W1, b1, ln_g, ln_b, W2, b2):
    raise NotImplementedError("write your pallas kernel here")


# speedup vs baseline: 2.7549x; 2.7549x over previous
import jax
import jax.numpy as jnp
from jax import lax
from jax.experimental import pallas as pl
from jax.experimental.pallas import tpu as pltpu
from jax.experimental.pallas import tpu_sc as plsc

_N = 10000
_E = 320000
_D = 128
_NC = 2
_NS = 16
_NW = _NC * _NS
_EPW = _E // _NW
_C = 80
_NCHUNK = _EPW // _C
_NBLK = _N // _C
_NBLK_PT = -(-_NBLK // _NS)
_BR = 1000
_BE = 2000


def _ln_rows(y, g, b):
    m = jnp.mean(y, axis=-1, keepdims=True)
    d = y - m
    v = jnp.mean(d * d, axis=-1, keepdims=True)
    return d * lax.rsqrt(v + 1e-5) * g + b


def _proj_body(h_ref, wT_ref, b_ref, g_ref, bb_ref, o_ref):
    x = h_ref[...]
    for i in range(2):
        y = jnp.dot(x[:, i * 64:(i + 1) * 64], wT_ref[i],
                    preferred_element_type=jnp.float32) + b_ref[i]
        y = _ln_rows(y, g_ref[i], bb_ref[i])
        o_ref[:, i * 64:(i + 1) * 64] = jnp.maximum(y, 0.0)


def _run_proj(h, proj_wT, proj_b, proj_ln_g, proj_ln_b):
    return pl.pallas_call(
        _proj_body,
        grid=(_N // _BR,),
        in_specs=[
            pl.BlockSpec((_BR, _D), lambda i: (i, 0)),
            pl.BlockSpec((2, 64, 64), lambda i: (0, 0, 0)),
            pl.BlockSpec((2, 64), lambda i: (0, 0)),
            pl.BlockSpec((2, 64), lambda i: (0, 0)),
            pl.BlockSpec((2, 64), lambda i: (0, 0)),
        ],
        out_specs=pl.BlockSpec((_BR, _D), lambda i: (i, 0)),
        out_shape=jax.ShapeDtypeStruct((_N, _D), jnp.float32),
    )(h, proj_wT, proj_b, proj_ln_g, proj_ln_b)


_sc_mesh = plsc.VectorSubcoreMesh(core_axis_name="c", subcore_axis_name="s")


def _mp_scatter_body(hh_hbm, src_hbm, dst_hbm, ew_hbm, out_hbm,
                     idx_s, idx_d, ew_v, rows, zbuf, acc, sem):
    c = lax.axis_index("c")
    s = lax.axis_index("s")
    wid = s * _NC + c

    z = jnp.zeros((16,), jnp.float32)

    def zrow(i, carry):
        for j in range(8):
            zbuf[i, pl.ds(j * 16, 16)] = z
        return carry

    lax.fori_loop(0, _C, zrow, 0)
    for k in range(_NBLK_PT):
        blk = s * _NBLK_PT + k

        @pl.when(blk < _NBLK)
        def _():
            pltpu.sync_copy(zbuf, acc.at[pl.ds(blk * _C, _C)])
    plsc.subcore_barrier()

    def chunk(i, carry):
        base = wid * _EPW + i * _C
        pltpu.sync_copy(src_hbm.at[pl.ds(base, _C)], idx_s)
        pltpu.sync_copy(dst_hbm.at[pl.ds(base, _C)], idx_d)
        pltpu.sync_copy(ew_hbm.at[pl.ds(base, _C)], ew_v.at[pl.ds(0, _C)])
        pltpu.async_copy(hh_hbm.at[idx_s], rows, sem).wait()

        def scale(r, cc):
            w = ew_v[pl.ds(r, 16)][0]
            for j in range(8):
                rows[r, pl.ds(j * 16, 16)] = rows[r, pl.ds(j * 16, 16)] * w
            return cc

        lax.fori_loop(0, _C, scale, 0, unroll=4)
        pltpu.sync_copy(rows, acc.at[idx_d], add=True)
        return carry

    lax.fori_loop(0, _NCHUNK, chunk, 0)
    plsc.subcore_barrier()
    for k in range(_NBLK_PT):
        blk = s * _NBLK_PT + k

        @pl.when(blk < _NBLK)
        def _():
            pltpu.sync_copy(acc.at[pl.ds(blk * _C, _C)],
                            out_hbm.at[c, pl.ds(blk * _C, _C)])


_mp_scatter = pl.kernel(
    _mp_scatter_body,
    out_type=jax.ShapeDtypeStruct((_NC, _N, _D), jnp.float32),
    mesh=_sc_mesh,
    scratch_types=[
        pltpu.VMEM((_C,), jnp.int32),
        pltpu.VMEM((_C,), jnp.int32),
        pltpu.VMEM((_C + 16,), jnp.float32),
        pltpu.VMEM((_C, _D), jnp.float32),
        pltpu.VMEM((_C, _D), jnp.float32),
        pltpu.VMEM_SHARED((_N, _D), jnp.float32),
        pltpu.SemaphoreType.DMA,
    ],
)


def _edge_gather_body(hh_hbm, src_hbm, dst_hbm, hu_hbm, hv_hbm,
                      idx_s, idx_d, rows_u, rows_v, sem_u, sem_v):
    c = lax.axis_index("c")
    s = lax.axis_index("s")
    wid = s * _NC + c

    def chunk(i, carry):
        base = wid * _EPW + i * _C
        pltpu.sync_copy(src_hbm.at[pl.ds(base, _C)], idx_s)
        pltpu.sync_copy(dst_hbm.at[pl.ds(base, _C)], idx_d)
        cu = pltpu.async_copy(hh_hbm.at[idx_s], rows_u, sem_u)
        cv = pltpu.async_copy(hh_hbm.at[idx_d], rows_v, sem_v)
        cu.wait()
        cv.wait()
        pltpu.sync_copy(rows_u, hu_hbm.at[pl.ds(base, _C)])
        pltpu.sync_copy(rows_v, hv_hbm.at[pl.ds(base, _C)])
        return carry

    lax.fori_loop(0, _NCHUNK, chunk, 0)


_edge_gather = pl.kernel(
    _edge_gather_body,
    out_type=(jax.ShapeDtypeStruct((_E, _D), jnp.float32),
              jax.ShapeDtypeStruct((_E, _D), jnp.float32)),
    mesh=_sc_mesh,
    scratch_types=[
        pltpu.VMEM((_C,), jnp.int32),
        pltpu.VMEM((_C,), jnp.int32),
        pltpu.VMEM((_C, _D), jnp.float32),
        pltpu.VMEM((_C, _D), jnp.float32),
        pltpu.SemaphoreType.DMA,
        pltpu.SemaphoreType.DMA,
    ],
)


def _mp_combine_body(hh_ref, a0_ref, a1_ref, n_ref, wTl_ref, wTr_ref,
                     b_ref, g_ref, bb_ref, o_ref):
    ah = (a0_ref[...] + a1_ref[...]) * n_ref[...]
    y = (jnp.dot(hh_ref[...], wTl_ref[...], preferred_element_type=jnp.float32)
         + jnp.dot(ah, wTr_ref[...], preferred_element_type=jnp.float32)
         + b_ref[...])
    y = _ln_rows(y, g_ref[...], bb_ref[...])
    o_ref[...] = jnp.maximum(y, 0.0)


def _run_mp_combine(hh, a0, a1, norm, wTl, wTr, b, g, bb):
    return pl.pallas_call(
        _mp_combine_body,
        grid=(_N // _BR,),
        in_specs=[
            pl.BlockSpec((_BR, _D), lambda i: (i, 0)),
            pl.BlockSpec((_BR, _D), lambda i: (i, 0)),
            pl.BlockSpec((_BR, _D), lambda i: (i, 0)),
            pl.BlockSpec((_BR, 1), lambda i: (i, 0)),
            pl.BlockSpec((_D, _D), lambda i: (0, 0)),
            pl.BlockSpec((_D, _D), lambda i: (0, 0)),
            pl.BlockSpec((_D,), lambda i: (0,)),
            pl.BlockSpec((_D,), lambda i: (0,)),
            pl.BlockSpec((_D,), lambda i: (0,)),
        ],
        out_specs=pl.BlockSpec((_BR, _D), lambda i: (i, 0)),
        out_shape=jax.ShapeDtypeStruct((_N, _D), jnp.float32),
    )(hh, a0, a1, norm, wTl, wTr, b, g, bb)


def _edge_mlp_body(hu_ref, hv_ref, ef_ref, w1u_ref, w1v_ref, b1_ref,
                   g_ref, bb_ref, w2a_ref, w2b_ref, b2_ref, o_ref):
    x = (jnp.dot(hu_ref[...], w1u_ref[...], preferred_element_type=jnp.float32)
         + jnp.dot(hv_ref[...], w1v_ref[...], preferred_element_type=jnp.float32)
         + b1_ref[...])
    x = jnp.maximum(_ln_rows(x, g_ref[...], bb_ref[...]), 0.0)
    sc = (jnp.dot(x, w2a_ref[...], preferred_element_type=jnp.float32)
          + jnp.dot(ef_ref[...], w2b_ref[...], preferred_element_type=jnp.float32)
          + b2_ref[...])
    o_ref[...] = sc


def _run_edge_mlp(hu, hv, ef, w1uT, w1vT, b1, g, bb, w2aT, w2bT, b2):
    return pl.pallas_call(
        _edge_mlp_body,
        grid=(_E // _BE,),
        in_specs=[
            pl.BlockSpec((_BE, _D), lambda i: (i, 0)),
            pl.BlockSpec((_BE, _D), lambda i: (i, 0)),
            pl.BlockSpec((_BE, 6), lambda i: (i, 0)),
            pl.BlockSpec((_D, 256), lambda i: (0, 0)),
            pl.BlockSpec((_D, 256), lambda i: (0, 0)),
            pl.BlockSpec((256,), lambda i: (0,)),
            pl.BlockSpec((256,), lambda i: (0,)),
            pl.BlockSpec((256,), lambda i: (0,)),
            pl.BlockSpec((256, 5), lambda i: (0, 0)),
            pl.BlockSpec((6, 5), lambda i: (0, 0)),
            pl.BlockSpec((5,), lambda i: (0,)),
        ],
        out_specs=pl.BlockSpec((_BE, 5), lambda i: (i, 0)),
        out_shape=jax.ShapeDtypeStruct((_E, 5), jnp.float32),
    )(hu, hv, ef, w1uT, w1vT, b1, g, bb, w2aT, w2bT, b2)


def kernel(h, edge_index, edge_w, norm, edge_feat, proj_w, proj_b, proj_ln_g,
           proj_ln_b, mp_w, mp_b, mp_ln_g, mp_ln_b, W1, b1, ln_g, ln_b, W2, b2):
    src = edge_index[0]
    dst = edge_index[1]
    ew = edge_w[:, 0]

    proj_wT = jnp.swapaxes(proj_w, 1, 2)
    hh = _run_proj(h, proj_wT, proj_b, proj_ln_g, proj_ln_b)

    for l in range(2):
        part = _mp_scatter(hh, src, dst, ew)
        hh = _run_mp_combine(hh, part[0], part[1], norm,
                             mp_w[l][:, :128].T, mp_w[l][:, 128:].T,
                             mp_b[l], mp_ln_g[l], mp_ln_b[l])

    hu, hv = _edge_gather(hh, src, dst)
    score = _run_edge_mlp(hu, hv, edge_feat,
                          W1[:, :128].T, W1[:, 128:].T, b1, ln_g, ln_b,
                          W2[:, :256].T, W2[:, 256:].T, b2)
    return score


# speedup vs baseline: 3.5959x; 1.3053x over previous
import jax
import jax.numpy as jnp
from jax import lax
from jax.experimental import pallas as pl
from jax.experimental.pallas import tpu as pltpu
from jax.experimental.pallas import tpu_sc as plsc

_N = 10000
_E = 320000
_D = 128
_NC = 2
_NS = 16
_NW = _NC * _NS
_EPW = _E // _NW
_C = 80
_NCHUNK = _EPW // _C
_NBLK = _N // _C
_NBLK_PT = -(-_NBLK // _NS)
_BR = 1000
_BE = 2000


def _ln_rows(y, g, b):
    m = jnp.mean(y, axis=-1, keepdims=True)
    d = y - m
    v = jnp.mean(d * d, axis=-1, keepdims=True)
    return d * lax.rsqrt(v + 1e-5) * g + b


def _proj_body(h_ref, wT_ref, b_ref, g_ref, bb_ref, o_ref):
    x = h_ref[...]
    for i in range(2):
        y = jnp.dot(x[:, i * 64:(i + 1) * 64], wT_ref[i],
                    preferred_element_type=jnp.float32) + b_ref[i]
        y = _ln_rows(y, g_ref[i], bb_ref[i])
        o_ref[:, i * 64:(i + 1) * 64] = jnp.maximum(y, 0.0)


def _run_proj(h, proj_wT, proj_b, proj_ln_g, proj_ln_b):
    return pl.pallas_call(
        _proj_body,
        grid=(_N // _BR,),
        in_specs=[
            pl.BlockSpec((_BR, _D), lambda i: (i, 0)),
            pl.BlockSpec((2, 64, 64), lambda i: (0, 0, 0)),
            pl.BlockSpec((2, 64), lambda i: (0, 0)),
            pl.BlockSpec((2, 64), lambda i: (0, 0)),
            pl.BlockSpec((2, 64), lambda i: (0, 0)),
        ],
        out_specs=pl.BlockSpec((_BR, _D), lambda i: (i, 0)),
        out_shape=jax.ShapeDtypeStruct((_N, _D), jnp.float32),
    )(h, proj_wT, proj_b, proj_ln_g, proj_ln_b)


_sc_mesh = plsc.VectorSubcoreMesh(core_axis_name="c", subcore_axis_name="s")


def _mp_scatter_body(hh_hbm, sd3_hbm, ew_hbm, out_hbm,
                     ib0, ib1, eb0, eb1, rows0, rows1, acc,
                     is0, is1, gs0, gs1):
    c = lax.axis_index("c")
    s = lax.axis_index("s")
    wid = s * _NC + c

    ibufs = ((ib0, eb0, is0), (ib1, eb1, is1))
    rbufs = ((rows0, gs0), (rows1, gs1))

    def idxload(i, b):
        ib, eb, sem = ibufs[b]
        base = wid * _EPW + i * _C
        return (pltpu.make_async_copy(sd3_hbm.at[wid, i], ib, sem),
                pltpu.make_async_copy(ew_hbm.at[pl.ds(base, _C)],
                                      eb.at[pl.ds(0, _C)], sem))

    def gather(i, b):
        ib = ibufs[b][0]
        rows, sem = rbufs[b]
        return pltpu.make_async_copy(hh_hbm.at[ib.at[0]], rows, sem)

    def process(i, b):
        ib, eb, _ = ibufs[b]
        rows, _g = rbufs[b]

        def scale(r, cc):
            w = eb[pl.ds(r, 16)][0]
            for j in range(8):
                rows[r, pl.ds(j * 16, 16)] = rows[r, pl.ds(j * 16, 16)] * w
            return cc

        lax.fori_loop(0, _C, scale, 0, unroll=4)
        pltpu.sync_copy(rows, acc.at[ib.at[1]], add=True)

    z = jnp.zeros((16,), jnp.float32)

    def zrow(i, carry):
        for j in range(8):
            rows0[i, pl.ds(j * 16, 16)] = z
        return carry

    lax.fori_loop(0, _C, zrow, 0)
    for k in range(_NBLK_PT):
        blk = s * _NBLK_PT + k

        @pl.when(blk < _NBLK)
        def _():
            pltpu.sync_copy(rows0, acc.at[pl.ds(blk * _C, _C)])

    for d in idxload(0, 0) + idxload(1, 1):
        d.start()
    plsc.subcore_barrier()
    for d in idxload(0, 0):
        d.wait()
    gather(0, 0).start()

    def pair(k, carry):
        g = k * 2
        for b in range(2):
            i = g + b
            gather(i, b).wait()
            for d in idxload(i + 1, 1 - b):
                d.wait()
            gather(i + 1, 1 - b).start()
            process(i, b)

            @pl.when(i + 2 < _NCHUNK)
            def _():
                for d in idxload(i + 2, b):
                    d.start()
        return carry

    lax.fori_loop(0, (_NCHUNK - 1) // 2, pair, 0)
    last = _NCHUNK - 1
    gather(last, last % 2).wait()
    process(last, last % 2)

    plsc.subcore_barrier()
    for k in range(_NBLK_PT):
        blk = s * _NBLK_PT + k

        @pl.when(blk < _NBLK)
        def _():
            pltpu.sync_copy(acc.at[pl.ds(blk * _C, _C)],
                            out_hbm.at[c, pl.ds(blk * _C, _C)])


_mp_scatter = pl.kernel(
    _mp_scatter_body,
    out_type=jax.ShapeDtypeStruct((_NC, _N, _D), jnp.float32),
    mesh=_sc_mesh,
    scratch_types=[
        pltpu.VMEM((2, _C), jnp.int32),
        pltpu.VMEM((2, _C), jnp.int32),
        pltpu.VMEM((_C + 16,), jnp.float32),
        pltpu.VMEM((_C + 16,), jnp.float32),
        pltpu.VMEM((_C, _D), jnp.float32),
        pltpu.VMEM((_C, _D), jnp.float32),
        pltpu.VMEM_SHARED((_N, _D), jnp.float32),
        pltpu.SemaphoreType.DMA,
        pltpu.SemaphoreType.DMA,
        pltpu.SemaphoreType.DMA,
        pltpu.SemaphoreType.DMA,
    ],
)


def _edge_gather_body(hh_hbm, src_hbm, dst_hbm, hu_hbm, hv_hbm,
                      idx_s, idx_d, rows_u, rows_v, sem_u, sem_v):
    c = lax.axis_index("c")
    s = lax.axis_index("s")
    wid = s * _NC + c

    def chunk(i, carry):
        base = wid * _EPW + i * _C
        pltpu.sync_copy(src_hbm.at[pl.ds(base, _C)], idx_s)
        pltpu.sync_copy(dst_hbm.at[pl.ds(base, _C)], idx_d)
        cu = pltpu.async_copy(hh_hbm.at[idx_s], rows_u, sem_u)
        cv = pltpu.async_copy(hh_hbm.at[idx_d], rows_v, sem_v)
        cu.wait()
        cv.wait()
        pltpu.sync_copy(rows_u, hu_hbm.at[pl.ds(base, _C)])
        pltpu.sync_copy(rows_v, hv_hbm.at[pl.ds(base, _C)])
        return carry

    lax.fori_loop(0, _NCHUNK, chunk, 0)


_edge_gather = pl.kernel(
    _edge_gather_body,
    out_type=(jax.ShapeDtypeStruct((_E, _D), jnp.float32),
              jax.ShapeDtypeStruct((_E, _D), jnp.float32)),
    mesh=_sc_mesh,
    scratch_types=[
        pltpu.VMEM((_C,), jnp.int32),
        pltpu.VMEM((_C,), jnp.int32),
        pltpu.VMEM((_C, _D), jnp.float32),
        pltpu.VMEM((_C, _D), jnp.float32),
        pltpu.SemaphoreType.DMA,
        pltpu.SemaphoreType.DMA,
    ],
)


def _mp_combine_body(hh_ref, a0_ref, a1_ref, n_ref, wTl_ref, wTr_ref,
                     b_ref, g_ref, bb_ref, o_ref):
    ah = (a0_ref[...] + a1_ref[...]) * n_ref[...]
    y = (jnp.dot(hh_ref[...], wTl_ref[...], preferred_element_type=jnp.float32)
         + jnp.dot(ah, wTr_ref[...], preferred_element_type=jnp.float32)
         + b_ref[...])
    y = _ln_rows(y, g_ref[...], bb_ref[...])
    o_ref[...] = jnp.maximum(y, 0.0)


def _run_mp_combine(hh, a0, a1, norm, wTl, wTr, b, g, bb):
    return pl.pallas_call(
        _mp_combine_body,
        grid=(_N // _BR,),
        in_specs=[
            pl.BlockSpec((_BR, _D), lambda i: (i, 0)),
            pl.BlockSpec((_BR, _D), lambda i: (i, 0)),
            pl.BlockSpec((_BR, _D), lambda i: (i, 0)),
            pl.BlockSpec((_BR, 1), lambda i: (i, 0)),
            pl.BlockSpec((_D, _D), lambda i: (0, 0)),
            pl.BlockSpec((_D, _D), lambda i: (0, 0)),
            pl.BlockSpec((_D,), lambda i: (0,)),
            pl.BlockSpec((_D,), lambda i: (0,)),
            pl.BlockSpec((_D,), lambda i: (0,)),
        ],
        out_specs=pl.BlockSpec((_BR, _D), lambda i: (i, 0)),
        out_shape=jax.ShapeDtypeStruct((_N, _D), jnp.float32),
    )(hh, a0, a1, norm, wTl, wTr, b, g, bb)


def _edge_mlp_body(hu_ref, hv_ref, ef_ref, w1u_ref, w1v_ref, b1_ref,
                   g_ref, bb_ref, w2a_ref, w2b_ref, b2_ref, o_ref):
    x = (jnp.dot(hu_ref[...], w1u_ref[...], preferred_element_type=jnp.float32)
         + jnp.dot(hv_ref[...], w1v_ref[...], preferred_element_type=jnp.float32)
         + b1_ref[...])
    x = jnp.maximum(_ln_rows(x, g_ref[...], bb_ref[...]), 0.0)
    sc = (jnp.dot(x, w2a_ref[...], preferred_element_type=jnp.float32)
          + jnp.dot(ef_ref[...], w2b_ref[...], preferred_element_type=jnp.float32)
          + b2_ref[...])
    o_ref[...] = sc


def _run_edge_mlp(hu, hv, ef, w1uT, w1vT, b1, g, bb, w2aT, w2bT, b2):
    return pl.pallas_call(
        _edge_mlp_body,
        grid=(_E // _BE,),
        in_specs=[
            pl.BlockSpec((_BE, _D), lambda i: (i, 0)),
            pl.BlockSpec((_BE, _D), lambda i: (i, 0)),
            pl.BlockSpec((_BE, 6), lambda i: (i, 0)),
            pl.BlockSpec((_D, 256), lambda i: (0, 0)),
            pl.BlockSpec((_D, 256), lambda i: (0, 0)),
            pl.BlockSpec((256,), lambda i: (0,)),
            pl.BlockSpec((256,), lambda i: (0,)),
            pl.BlockSpec((256,), lambda i: (0,)),
            pl.BlockSpec((256, 5), lambda i: (0, 0)),
            pl.BlockSpec((6, 5), lambda i: (0, 0)),
            pl.BlockSpec((5,), lambda i: (0,)),
        ],
        out_specs=pl.BlockSpec((_BE, 5), lambda i: (i, 0)),
        out_shape=jax.ShapeDtypeStruct((_E, 5), jnp.float32),
    )(hu, hv, ef, w1uT, w1vT, b1, g, bb, w2aT, w2bT, b2)


def kernel(h, edge_index, edge_w, norm, edge_feat, proj_w, proj_b, proj_ln_g,
           proj_ln_b, mp_w, mp_b, mp_ln_g, mp_ln_b, W1, b1, ln_g, ln_b, W2, b2):
    src = edge_index[0]
    dst = edge_index[1]
    sd3 = jnp.stack([src.reshape(_NW, _NCHUNK, _C),
                     dst.reshape(_NW, _NCHUNK, _C)], axis=2)
    ew = edge_w[:, 0]

    proj_wT = jnp.swapaxes(proj_w, 1, 2)
    hh = _run_proj(h, proj_wT, proj_b, proj_ln_g, proj_ln_b)

    for l in range(2):
        part = _mp_scatter(hh, sd3, ew)
        hh = _run_mp_combine(hh, part[0], part[1], norm,
                             mp_w[l][:, :128].T, mp_w[l][:, 128:].T,
                             mp_b[l], mp_ln_g[l], mp_ln_b[l])

    hu, hv = _edge_gather(hh, src, dst)
    score = _run_edge_mlp(hu, hv, edge_feat,
                          W1[:, :128].T, W1[:, 128:].T, b1, ln_g, ln_b,
                          W2[:, :256].T, W2[:, 256:].T, b2)
    return score


# speedup vs baseline: 4.4060x; 1.2253x over previous
import jax
import jax.numpy as jnp
from jax import lax
from jax.experimental import pallas as pl
from jax.experimental.pallas import tpu as pltpu
from jax.experimental.pallas import tpu_sc as plsc

_N = 10000
_E = 320000
_D = 128
_NC = 2
_NS = 16
_NW = _NC * _NS
_EPW = _E // _NW
_C = 80
_NCHUNK = _EPW // _C
_NBLK = _N // _C
_NBLK_PT = -(-_NBLK // _NS)
_BR = 1000
_BE = 2000


def _ln_rows(y, g, b):
    m = jnp.mean(y, axis=-1, keepdims=True)
    d = y - m
    v = jnp.mean(d * d, axis=-1, keepdims=True)
    return d * lax.rsqrt(v + 1e-5) * g + b


def _proj_body(h_ref, wT_ref, b_ref, g_ref, bb_ref, o_ref):
    x = h_ref[...]
    for i in range(2):
        y = jnp.dot(x[:, i * 64:(i + 1) * 64], wT_ref[i],
                    preferred_element_type=jnp.float32) + b_ref[i]
        y = _ln_rows(y, g_ref[i], bb_ref[i])
        o_ref[:, i * 64:(i + 1) * 64] = jnp.maximum(y, 0.0)


def _run_proj(h, proj_wT, proj_b, proj_ln_g, proj_ln_b):
    return pl.pallas_call(
        _proj_body,
        grid=(_N // _BR,),
        in_specs=[
            pl.BlockSpec((_BR, _D), lambda i: (i, 0)),
            pl.BlockSpec((2, 64, 64), lambda i: (0, 0, 0)),
            pl.BlockSpec((2, 64), lambda i: (0, 0)),
            pl.BlockSpec((2, 64), lambda i: (0, 0)),
            pl.BlockSpec((2, 64), lambda i: (0, 0)),
        ],
        out_specs=pl.BlockSpec((_BR, _D), lambda i: (i, 0)),
        out_shape=jax.ShapeDtypeStruct((_N, _D), jnp.float32),
    )(h, proj_wT, proj_b, proj_ln_g, proj_ln_b)


_sc_mesh = plsc.VectorSubcoreMesh(core_axis_name="c", subcore_axis_name="s")


def _mp_scatter_body(hh_hbm, sd3_hbm, ew_hbm, out_hbm,
                     ib0, ib1, eb0, eb1, rows0, rows1, acc,
                     is0, is1, gs0, gs1):
    c = lax.axis_index("c")
    s = lax.axis_index("s")
    wid = s * _NC + c

    ibufs = ((ib0, eb0, is0), (ib1, eb1, is1))
    rbufs = ((rows0, gs0), (rows1, gs1))

    def idxload(i, b):
        ib, eb, sem = ibufs[b]
        base = wid * _EPW + i * _C
        return (pltpu.make_async_copy(sd3_hbm.at[wid, i], ib, sem),
                pltpu.make_async_copy(ew_hbm.at[pl.ds(base, _C)],
                                      eb.at[pl.ds(0, _C)], sem))

    def gather(i, b):
        ib = ibufs[b][0]
        rows, sem = rbufs[b]
        return pltpu.make_async_copy(hh_hbm.at[ib.at[0]], rows, sem)

    def process(i, b):
        ib, eb, _ = ibufs[b]
        rows, _g = rbufs[b]

        def scale(r, cc):
            w = eb[pl.ds(r, 16)][0]
            for j in range(8):
                rows[r, pl.ds(j * 16, 16)] = rows[r, pl.ds(j * 16, 16)] * w
            return cc

        lax.fori_loop(0, _C, scale, 0, unroll=4)
        pltpu.sync_copy(rows, acc.at[ib.at[1]], add=True)

    z = jnp.zeros((16,), jnp.float32)

    def zrow(i, carry):
        for j in range(8):
            rows0[i, pl.ds(j * 16, 16)] = z
        return carry

    lax.fori_loop(0, _C, zrow, 0)
    for k in range(_NBLK_PT):
        blk = s * _NBLK_PT + k

        @pl.when(blk < _NBLK)
        def _():
            pltpu.sync_copy(rows0, acc.at[pl.ds(blk * _C, _C)])

    for d in idxload(0, 0) + idxload(1, 1):
        d.start()
    plsc.subcore_barrier()
    for d in idxload(0, 0):
        d.wait()
    gather(0, 0).start()

    def pair(k, carry):
        g = k * 2
        for b in range(2):
            i = g + b
            gather(i, b).wait()
            for d in idxload(i + 1, 1 - b):
                d.wait()
            gather(i + 1, 1 - b).start()
            process(i, b)

            @pl.when(i + 2 < _NCHUNK)
            def _():
                for d in idxload(i + 2, b):
                    d.start()
        return carry

    lax.fori_loop(0, (_NCHUNK - 1) // 2, pair, 0)
    last = _NCHUNK - 1
    gather(last, last % 2).wait()
    process(last, last % 2)

    plsc.subcore_barrier()
    for k in range(_NBLK_PT):
        blk = s * _NBLK_PT + k

        @pl.when(blk < _NBLK)
        def _():
            pltpu.sync_copy(acc.at[pl.ds(blk * _C, _C)],
                            out_hbm.at[c, pl.ds(blk * _C, _C)])


_mp_scatter = pl.kernel(
    _mp_scatter_body,
    out_type=jax.ShapeDtypeStruct((_NC, _N, _D), jnp.float32),
    mesh=_sc_mesh,
    scratch_types=[
        pltpu.VMEM((2, _C), jnp.int32),
        pltpu.VMEM((2, _C), jnp.int32),
        pltpu.VMEM((_C + 16,), jnp.float32),
        pltpu.VMEM((_C + 16,), jnp.float32),
        pltpu.VMEM((_C, _D), jnp.float32),
        pltpu.VMEM((_C, _D), jnp.float32),
        pltpu.VMEM_SHARED((_N, _D), jnp.float32),
        pltpu.SemaphoreType.DMA,
        pltpu.SemaphoreType.DMA,
        pltpu.SemaphoreType.DMA,
        pltpu.SemaphoreType.DMA,
    ],
)


def _edge_gather_body(hh_hbm, sd3_hbm, hu_hbm, hv_hbm,
                      ib0, ib1, ru0, ru1, rv0, rv1, hh_s,
                      is0, is1, wu0, wu1, wv0, wv1):
    c = lax.axis_index("c")
    s = lax.axis_index("s")
    wid = s * _NC + c

    ibufs = ((ib0, is0), (ib1, is1))
    rbufs = ((ru0, rv0, wu0, wv0), (ru1, rv1, wu1, wv1))

    def idxload(i, b):
        ib, sem = ibufs[b]
        return pltpu.make_async_copy(sd3_hbm.at[wid, i], ib, sem)

    def writes(i, b):
        ru, rv, wu, wv = rbufs[b]
        base = wid * _EPW + i * _C
        return (pltpu.make_async_copy(ru, hu_hbm.at[pl.ds(base, _C)], wu),
                pltpu.make_async_copy(rv, hv_hbm.at[pl.ds(base, _C)], wv))

    idxload(0, 0).start()
    idxload(1, 1).start()
    for k in range(_NBLK_PT):
        blk = s * _NBLK_PT + k

        @pl.when(blk < _NBLK)
        def _():
            pltpu.sync_copy(hh_hbm.at[pl.ds(blk * _C, _C)],
                            hh_s.at[pl.ds(blk * _C, _C)])
    plsc.subcore_barrier()

    def pair(k, carry):
        g = k * 2
        for b in range(2):
            i = g + b
            ib, _ = ibufs[b]
            ru, rv, _wu, _wv = rbufs[b]
            idxload(i, b).wait()

            @pl.when(i >= 2)
            def _():
                for d in writes(i - 2, b):
                    d.wait()

            pltpu.sync_copy(hh_s.at[ib.at[0]], ru)
            pltpu.sync_copy(hh_s.at[ib.at[1]], rv)

            @pl.when(i + 2 < _NCHUNK)
            def _():
                idxload(i + 2, b).start()

            for d in writes(i, b):
                d.start()
        return carry

    lax.fori_loop(0, (_NCHUNK - 1) // 2, pair, 0)

    last = _NCHUNK - 1
    lb = last % 2
    ib, _ = ibufs[lb]
    ru, rv, _wu, _wv = rbufs[lb]
    idxload(last, lb).wait()
    for d in writes(last - 2, lb):
        d.wait()
    pltpu.sync_copy(hh_s.at[ib.at[0]], ru)
    pltpu.sync_copy(hh_s.at[ib.at[1]], rv)
    for d in writes(last, lb):
        d.start()
    for d in writes(last - 1, 1 - lb) + writes(last, lb):
        d.wait()


_edge_gather = pl.kernel(
    _edge_gather_body,
    out_type=(jax.ShapeDtypeStruct((_E, _D), jnp.float32),
              jax.ShapeDtypeStruct((_E, _D), jnp.float32)),
    mesh=_sc_mesh,
    scratch_types=[
        pltpu.VMEM((2, _C), jnp.int32),
        pltpu.VMEM((2, _C), jnp.int32),
        pltpu.VMEM((_C, _D), jnp.float32),
        pltpu.VMEM((_C, _D), jnp.float32),
        pltpu.VMEM((_C, _D), jnp.float32),
        pltpu.VMEM((_C, _D), jnp.float32),
        pltpu.VMEM_SHARED((_N, _D), jnp.float32),
        pltpu.SemaphoreType.DMA,
        pltpu.SemaphoreType.DMA,
        pltpu.SemaphoreType.DMA,
        pltpu.SemaphoreType.DMA,
        pltpu.SemaphoreType.DMA,
        pltpu.SemaphoreType.DMA,
    ],
)


def _mp_combine_body(hh_ref, a0_ref, a1_ref, n_ref, wTl_ref, wTr_ref,
                     b_ref, g_ref, bb_ref, o_ref):
    ah = (a0_ref[...] + a1_ref[...]) * n_ref[...]
    y = (jnp.dot(hh_ref[...], wTl_ref[...], preferred_element_type=jnp.float32)
         + jnp.dot(ah, wTr_ref[...], preferred_element_type=jnp.float32)
         + b_ref[...])
    y = _ln_rows(y, g_ref[...], bb_ref[...])
    o_ref[...] = jnp.maximum(y, 0.0)


def _run_mp_combine(hh, a0, a1, norm, wTl, wTr, b, g, bb):
    return pl.pallas_call(
        _mp_combine_body,
        grid=(_N // _BR,),
        in_specs=[
            pl.BlockSpec((_BR, _D), lambda i: (i, 0)),
            pl.BlockSpec((_BR, _D), lambda i: (i, 0)),
            pl.BlockSpec((_BR, _D), lambda i: (i, 0)),
            pl.BlockSpec((_BR, 1), lambda i: (i, 0)),
            pl.BlockSpec((_D, _D), lambda i: (0, 0)),
            pl.BlockSpec((_D, _D), lambda i: (0, 0)),
            pl.BlockSpec((_D,), lambda i: (0,)),
            pl.BlockSpec((_D,), lambda i: (0,)),
            pl.BlockSpec((_D,), lambda i: (0,)),
        ],
        out_specs=pl.BlockSpec((_BR, _D), lambda i: (i, 0)),
        out_shape=jax.ShapeDtypeStruct((_N, _D), jnp.float32),
    )(hh, a0, a1, norm, wTl, wTr, b, g, bb)


def _edge_mlp_body(hu_ref, hv_ref, ef_ref, w1u_ref, w1v_ref, b1_ref,
                   g_ref, bb_ref, w2a_ref, w2b_ref, b2_ref, o_ref):
    x = (jnp.dot(hu_ref[...], w1u_ref[...], preferred_element_type=jnp.float32)
         + jnp.dot(hv_ref[...], w1v_ref[...], preferred_element_type=jnp.float32)
         + b1_ref[...])
    x = jnp.maximum(_ln_rows(x, g_ref[...], bb_ref[...]), 0.0)
    sc = (jnp.dot(x, w2a_ref[...], preferred_element_type=jnp.float32)
          + jnp.dot(ef_ref[...], w2b_ref[...], preferred_element_type=jnp.float32)
          + b2_ref[...])
    o_ref[...] = sc


def _run_edge_mlp(hu, hv, ef, w1uT, w1vT, b1, g, bb, w2aT, w2bT, b2):
    return pl.pallas_call(
        _edge_mlp_body,
        grid=(_E // _BE,),
        in_specs=[
            pl.BlockSpec((_BE, _D), lambda i: (i, 0)),
            pl.BlockSpec((_BE, _D), lambda i: (i, 0)),
            pl.BlockSpec((_BE, 6), lambda i: (i, 0)),
            pl.BlockSpec((_D, 256), lambda i: (0, 0)),
            pl.BlockSpec((_D, 256), lambda i: (0, 0)),
            pl.BlockSpec((256,), lambda i: (0,)),
            pl.BlockSpec((256,), lambda i: (0,)),
            pl.BlockSpec((256,), lambda i: (0,)),
            pl.BlockSpec((256, 5), lambda i: (0, 0)),
            pl.BlockSpec((6, 5), lambda i: (0, 0)),
            pl.BlockSpec((5,), lambda i: (0,)),
        ],
        out_specs=pl.BlockSpec((_BE, 5), lambda i: (i, 0)),
        out_shape=jax.ShapeDtypeStruct((_E, 5), jnp.float32),
    )(hu, hv, ef, w1uT, w1vT, b1, g, bb, w2aT, w2bT, b2)


def kernel(h, edge_index, edge_w, norm, edge_feat, proj_w, proj_b, proj_ln_g,
           proj_ln_b, mp_w, mp_b, mp_ln_g, mp_ln_b, W1, b1, ln_g, ln_b, W2, b2):
    src = edge_index[0]
    dst = edge_index[1]
    sd3 = jnp.stack([src.reshape(_NW, _NCHUNK, _C),
                     dst.reshape(_NW, _NCHUNK, _C)], axis=2)
    ew = edge_w[:, 0]

    proj_wT = jnp.swapaxes(proj_w, 1, 2)
    hh = _run_proj(h, proj_wT, proj_b, proj_ln_g, proj_ln_b)

    for l in range(2):
        part = _mp_scatter(hh, sd3, ew)
        hh = _run_mp_combine(hh, part[0], part[1], norm,
                             mp_w[l][:, :128].T, mp_w[l][:, 128:].T,
                             mp_b[l], mp_ln_g[l], mp_ln_b[l])

    hu, hv = _edge_gather(hh, sd3)
    score = _run_edge_mlp(hu, hv, edge_feat,
                          W1[:, :128].T, W1[:, 128:].T, b1, ln_g, ln_b,
                          W2[:, :256].T, W2[:, 256:].T, b2)
    return score


# speedup vs baseline: 4.7253x; 1.0725x over previous
import jax
import jax.numpy as jnp
from jax import lax
from jax.experimental import pallas as pl
from jax.experimental.pallas import tpu as pltpu
from jax.experimental.pallas import tpu_sc as plsc

_N = 10000
_E = 320000
_D = 128
_NC = 2
_NS = 16
_NW = _NC * _NS
_EPW = _E // _NW
_C = 80
_NCHUNK = _EPW // _C
_NBLK = _N // _C
_NBLK_PT = -(-_NBLK // _NS)
_BR = 1000
_BE = 2000


def _ln_rows(y, g, b):
    m = jnp.mean(y, axis=-1, keepdims=True)
    d = y - m
    v = jnp.mean(d * d, axis=-1, keepdims=True)
    return d * lax.rsqrt(v + 1e-5) * g + b


def _proj_body(h_ref, wT_ref, b_ref, g_ref, bb_ref, o_ref):
    x = h_ref[...]
    for i in range(2):
        y = jnp.dot(x[:, i * 64:(i + 1) * 64], wT_ref[i],
                    preferred_element_type=jnp.float32) + b_ref[i]
        y = _ln_rows(y, g_ref[i], bb_ref[i])
        o_ref[:, i * 64:(i + 1) * 64] = jnp.maximum(y, 0.0)


def _run_proj(h, proj_wT, proj_b, proj_ln_g, proj_ln_b):
    return pl.pallas_call(
        _proj_body,
        grid=(_N // _BR,),
        in_specs=[
            pl.BlockSpec((_BR, _D), lambda i: (i, 0)),
            pl.BlockSpec((2, 64, 64), lambda i: (0, 0, 0)),
            pl.BlockSpec((2, 64), lambda i: (0, 0)),
            pl.BlockSpec((2, 64), lambda i: (0, 0)),
            pl.BlockSpec((2, 64), lambda i: (0, 0)),
        ],
        out_specs=pl.BlockSpec((_BR, _D), lambda i: (i, 0)),
        out_shape=jax.ShapeDtypeStruct((_N, _D), jnp.float32),
    )(h, proj_wT, proj_b, proj_ln_g, proj_ln_b)


_sc_mesh = plsc.VectorSubcoreMesh(core_axis_name="c", subcore_axis_name="s")


def _mp_scatter_body(hh_hbm, sd3_hbm, ew_hbm, out_hbm,
                     ib0, ib1, eb0, eb1, rows0, rows1, acc,
                     is0, is1, gs0, gs1):
    c = lax.axis_index("c")
    s = lax.axis_index("s")
    wid = s * _NC + c

    ibufs = ((ib0, eb0, is0), (ib1, eb1, is1))
    rbufs = ((rows0, gs0), (rows1, gs1))

    def idxload(i, b):
        ib, eb, sem = ibufs[b]
        base = wid * _EPW + i * _C
        return (pltpu.make_async_copy(sd3_hbm.at[wid, i], ib, sem),
                pltpu.make_async_copy(ew_hbm.at[pl.ds(base, _C)],
                                      eb.at[pl.ds(0, _C)], sem))

    def gather(i, b):
        ib = ibufs[b][0]
        rows, sem = rbufs[b]
        return pltpu.make_async_copy(hh_hbm.at[ib.at[0]], rows, sem)

    def process(i, b):
        ib, eb, _ = ibufs[b]
        rows, _g = rbufs[b]

        @plsc.parallel_loop(0, _C, unroll=8)
        def _scale(r):
            w = eb[pl.ds(r, 16)][0]
            for j in range(8):
                rows[r, pl.ds(j * 16, 16)] = rows[r, pl.ds(j * 16, 16)] * w

        pltpu.sync_copy(rows, acc.at[ib.at[1]], add=True)

    z = jnp.zeros((16,), jnp.float32)

    def zrow(i, carry):
        for j in range(8):
            rows0[i, pl.ds(j * 16, 16)] = z
        return carry

    lax.fori_loop(0, _C, zrow, 0)
    for k in range(_NBLK_PT):
        blk = s * _NBLK_PT + k

        @pl.when(blk < _NBLK)
        def _():
            pltpu.sync_copy(rows0, acc.at[pl.ds(blk * _C, _C)])

    for d in idxload(0, 0) + idxload(1, 1):
        d.start()
    plsc.subcore_barrier()
    for d in idxload(0, 0):
        d.wait()
    gather(0, 0).start()

    def pair(k, carry):
        g = k * 2
        for b in range(2):
            i = g + b
            gather(i, b).wait()
            for d in idxload(i + 1, 1 - b):
                d.wait()
            gather(i + 1, 1 - b).start()
            process(i, b)

            @pl.when(i + 2 < _NCHUNK)
            def _():
                for d in idxload(i + 2, b):
                    d.start()
        return carry

    lax.fori_loop(0, (_NCHUNK - 1) // 2, pair, 0)
    last = _NCHUNK - 1
    gather(last, last % 2).wait()
    process(last, last % 2)

    plsc.subcore_barrier()
    for k in range(_NBLK_PT):
        blk = s * _NBLK_PT + k

        @pl.when(blk < _NBLK)
        def _():
            pltpu.sync_copy(acc.at[pl.ds(blk * _C, _C)],
                            out_hbm.at[c, pl.ds(blk * _C, _C)])


_mp_scatter = pl.kernel(
    _mp_scatter_body,
    out_type=jax.ShapeDtypeStruct((_NC, _N, _D), jnp.float32),
    mesh=_sc_mesh,
    scratch_types=[
        pltpu.VMEM((2, _C), jnp.int32),
        pltpu.VMEM((2, _C), jnp.int32),
        pltpu.VMEM((_C + 16,), jnp.float32),
        pltpu.VMEM((_C + 16,), jnp.float32),
        pltpu.VMEM((_C, _D), jnp.float32),
        pltpu.VMEM((_C, _D), jnp.float32),
        pltpu.VMEM_SHARED((_N, _D), jnp.float32),
        pltpu.SemaphoreType.DMA,
        pltpu.SemaphoreType.DMA,
        pltpu.SemaphoreType.DMA,
        pltpu.SemaphoreType.DMA,
    ],
)


def _edge_gather_body(hh_hbm, sd3_hbm, hu_hbm, hv_hbm,
                      ib0, ib1, ru0, ru1, rv0, rv1, hh_s,
                      is0, is1, wu0, wu1, wv0, wv1):
    c = lax.axis_index("c")
    s = lax.axis_index("s")
    wid = s * _NC + c

    ibufs = ((ib0, is0), (ib1, is1))
    rbufs = ((ru0, rv0, wu0, wv0), (ru1, rv1, wu1, wv1))

    def idxload(i, b):
        ib, sem = ibufs[b]
        return pltpu.make_async_copy(sd3_hbm.at[wid, i], ib, sem)

    def writes(i, b):
        ru, rv, wu, wv = rbufs[b]
        base = wid * _EPW + i * _C
        return (pltpu.make_async_copy(ru, hu_hbm.at[pl.ds(base, _C)], wu),
                pltpu.make_async_copy(rv, hv_hbm.at[pl.ds(base, _C)], wv))

    idxload(0, 0).start()
    idxload(1, 1).start()
    for k in range(_NBLK_PT):
        blk = s * _NBLK_PT + k

        @pl.when(blk < _NBLK)
        def _():
            pltpu.sync_copy(hh_hbm.at[pl.ds(blk * _C, _C)],
                            hh_s.at[pl.ds(blk * _C, _C)])
    plsc.subcore_barrier()

    def pair(k, carry):
        g = k * 2
        for b in range(2):
            i = g + b
            ib, _ = ibufs[b]
            ru, rv, _wu, _wv = rbufs[b]
            idxload(i, b).wait()

            @pl.when(i >= 2)
            def _():
                for d in writes(i - 2, b):
                    d.wait()

            pltpu.sync_copy(hh_s.at[ib.at[0]], ru)
            pltpu.sync_copy(hh_s.at[ib.at[1]], rv)

            @pl.when(i + 2 < _NCHUNK)
            def _():
                idxload(i + 2, b).start()

            for d in writes(i, b):
                d.start()
        return carry

    lax.fori_loop(0, (_NCHUNK - 1) // 2, pair, 0)

    last = _NCHUNK - 1
    lb = last % 2
    ib, _ = ibufs[lb]
    ru, rv, _wu, _wv = rbufs[lb]
    idxload(last, lb).wait()
    for d in writes(last - 2, lb):
        d.wait()
    pltpu.sync_copy(hh_s.at[ib.at[0]], ru)
    pltpu.sync_copy(hh_s.at[ib.at[1]], rv)
    for d in writes(last, lb):
        d.start()
    for d in writes(last - 1, 1 - lb) + writes(last, lb):
        d.wait()


_edge_gather = pl.kernel(
    _edge_gather_body,
    out_type=(jax.ShapeDtypeStruct((_E, _D), jnp.float32),
              jax.ShapeDtypeStruct((_E, _D), jnp.float32)),
    mesh=_sc_mesh,
    scratch_types=[
        pltpu.VMEM((2, _C), jnp.int32),
        pltpu.VMEM((2, _C), jnp.int32),
        pltpu.VMEM((_C, _D), jnp.float32),
        pltpu.VMEM((_C, _D), jnp.float32),
        pltpu.VMEM((_C, _D), jnp.float32),
        pltpu.VMEM((_C, _D), jnp.float32),
        pltpu.VMEM_SHARED((_N, _D), jnp.float32),
        pltpu.SemaphoreType.DMA,
        pltpu.SemaphoreType.DMA,
        pltpu.SemaphoreType.DMA,
        pltpu.SemaphoreType.DMA,
        pltpu.SemaphoreType.DMA,
        pltpu.SemaphoreType.DMA,
    ],
)


def _mp_combine_body(hh_ref, a0_ref, a1_ref, n_ref, wTl_ref, wTr_ref,
                     b_ref, g_ref, bb_ref, o_ref):
    ah = (a0_ref[...] + a1_ref[...]) * n_ref[...]
    y = (jnp.dot(hh_ref[...], wTl_ref[...], preferred_element_type=jnp.float32)
         + jnp.dot(ah, wTr_ref[...], preferred_element_type=jnp.float32)
         + b_ref[...])
    y = _ln_rows(y, g_ref[...], bb_ref[...])
    o_ref[...] = jnp.maximum(y, 0.0)


def _run_mp_combine(hh, a0, a1, norm, wTl, wTr, b, g, bb):
    return pl.pallas_call(
        _mp_combine_body,
        grid=(_N // _BR,),
        in_specs=[
            pl.BlockSpec((_BR, _D), lambda i: (i, 0)),
            pl.BlockSpec((_BR, _D), lambda i: (i, 0)),
            pl.BlockSpec((_BR, _D), lambda i: (i, 0)),
            pl.BlockSpec((_BR, 1), lambda i: (i, 0)),
            pl.BlockSpec((_D, _D), lambda i: (0, 0)),
            pl.BlockSpec((_D, _D), lambda i: (0, 0)),
            pl.BlockSpec((_D,), lambda i: (0,)),
            pl.BlockSpec((_D,), lambda i: (0,)),
            pl.BlockSpec((_D,), lambda i: (0,)),
        ],
        out_specs=pl.BlockSpec((_BR, _D), lambda i: (i, 0)),
        out_shape=jax.ShapeDtypeStruct((_N, _D), jnp.float32),
    )(hh, a0, a1, norm, wTl, wTr, b, g, bb)


def _edge_mlp_body(hu_ref, hv_ref, ef_ref, w1u_ref, w1v_ref, b1_ref,
                   g_ref, bb_ref, w2a_ref, w2b_ref, b2_ref, o_ref):
    x = (jnp.dot(hu_ref[...], w1u_ref[...], preferred_element_type=jnp.float32)
         + jnp.dot(hv_ref[...], w1v_ref[...], preferred_element_type=jnp.float32)
         + b1_ref[...])
    x = jnp.maximum(_ln_rows(x, g_ref[...], bb_ref[...]), 0.0)
    sc = (jnp.dot(x, w2a_ref[...], preferred_element_type=jnp.float32)
          + jnp.dot(ef_ref[...], w2b_ref[...], preferred_element_type=jnp.float32)
          + b2_ref[...])
    o_ref[...] = sc


def _run_edge_mlp(hu, hv, ef, w1uT, w1vT, b1, g, bb, w2aT, w2bT, b2):
    return pl.pallas_call(
        _edge_mlp_body,
        grid=(_E // _BE,),
        in_specs=[
            pl.BlockSpec((_BE, _D), lambda i: (i, 0)),
            pl.BlockSpec((_BE, _D), lambda i: (i, 0)),
            pl.BlockSpec((_BE, 6), lambda i: (i, 0)),
            pl.BlockSpec((_D, 256), lambda i: (0, 0)),
            pl.BlockSpec((_D, 256), lambda i: (0, 0)),
            pl.BlockSpec((256,), lambda i: (0,)),
            pl.BlockSpec((256,), lambda i: (0,)),
            pl.BlockSpec((256,), lambda i: (0,)),
            pl.BlockSpec((256, 5), lambda i: (0, 0)),
            pl.BlockSpec((6, 5), lambda i: (0, 0)),
            pl.BlockSpec((5,), lambda i: (0,)),
        ],
        out_specs=pl.BlockSpec((_BE, 5), lambda i: (i, 0)),
        out_shape=jax.ShapeDtypeStruct((_E, 5), jnp.float32),
    )(hu, hv, ef, w1uT, w1vT, b1, g, bb, w2aT, w2bT, b2)


def kernel(h, edge_index, edge_w, norm, edge_feat, proj_w, proj_b, proj_ln_g,
           proj_ln_b, mp_w, mp_b, mp_ln_g, mp_ln_b, W1, b1, ln_g, ln_b, W2, b2):
    src = edge_index[0]
    dst = edge_index[1]
    sd3 = jnp.stack([src.reshape(_NW, _NCHUNK, _C),
                     dst.reshape(_NW, _NCHUNK, _C)], axis=2)
    ew = edge_w[:, 0]

    proj_wT = jnp.swapaxes(proj_w, 1, 2)
    hh = _run_proj(h, proj_wT, proj_b, proj_ln_g, proj_ln_b)

    for l in range(2):
        part = _mp_scatter(hh, sd3, ew)
        hh = _run_mp_combine(hh, part[0], part[1], norm,
                             mp_w[l][:, :128].T, mp_w[l][:, 128:].T,
                             mp_b[l], mp_ln_g[l], mp_ln_b[l])

    hu, hv = _edge_gather(hh, sd3)
    score = _run_edge_mlp(hu, hv, edge_feat,
                          W1[:, :128].T, W1[:, 128:].T, b1, ln_g, ln_b,
                          W2[:, :256].T, W2[:, 256:].T, b2)
    return score


# speedup vs baseline: 4.7788x; 1.0113x over previous
import jax
import jax.numpy as jnp
from jax import lax
from jax.experimental import pallas as pl
from jax.experimental.pallas import tpu as pltpu
from jax.experimental.pallas import tpu_sc as plsc

_N = 10000
_E = 320000
_D = 128
_NC = 2
_NS = 16
_NW = _NC * _NS
_EPW = _E // _NW
_C = 80
_NCHUNK = _EPW // _C
_NBLK = _N // _C
_NBLK_PT = -(-_NBLK // _NS)
_BR = 1000
_BE = 2000


def _ln_rows(y, g, b):
    m = jnp.mean(y, axis=-1, keepdims=True)
    d = y - m
    v = jnp.mean(d * d, axis=-1, keepdims=True)
    return d * lax.rsqrt(v + 1e-5) * g + b


def _proj_body(h_ref, wT_ref, b_ref, g_ref, bb_ref, o_ref):
    x = h_ref[...]
    for i in range(2):
        y = jnp.dot(x[:, i * 64:(i + 1) * 64], wT_ref[i],
                    preferred_element_type=jnp.float32) + b_ref[i]
        y = _ln_rows(y, g_ref[i], bb_ref[i])
        o_ref[:, i * 64:(i + 1) * 64] = jnp.maximum(y, 0.0)


def _run_proj(h, proj_wT, proj_b, proj_ln_g, proj_ln_b):
    return pl.pallas_call(
        _proj_body,
        grid=(_N // _BR,),
        in_specs=[
            pl.BlockSpec((_BR, _D), lambda i: (i, 0)),
            pl.BlockSpec((2, 64, 64), lambda i: (0, 0, 0)),
            pl.BlockSpec((2, 64), lambda i: (0, 0)),
            pl.BlockSpec((2, 64), lambda i: (0, 0)),
            pl.BlockSpec((2, 64), lambda i: (0, 0)),
        ],
        out_specs=pl.BlockSpec((_BR, _D), lambda i: (i, 0)),
        out_shape=jax.ShapeDtypeStruct((_N, _D), jnp.float32),
    )(h, proj_wT, proj_b, proj_ln_g, proj_ln_b)


_sc_mesh = plsc.VectorSubcoreMesh(core_axis_name="c", subcore_axis_name="s")


def _mp_scatter_body(hh_hbm, sd3_hbm, ew_hbm, out_hbm,
                     ib0, ib1, eb0, eb1, rows0, rows1, acc,
                     is0, is1, gs0, gs1):
    c = lax.axis_index("c")
    s = lax.axis_index("s")
    wid = s * _NC + c

    ibufs = ((ib0, eb0, is0), (ib1, eb1, is1))
    rbufs = ((rows0, gs0), (rows1, gs1))

    def idxload(i, b):
        ib, eb, sem = ibufs[b]
        base = wid * _EPW + i * _C
        return (pltpu.make_async_copy(sd3_hbm.at[wid, i], ib, sem),
                pltpu.make_async_copy(ew_hbm.at[pl.ds(base, _C)],
                                      eb.at[pl.ds(0, _C)], sem))

    def gather(i, b):
        ib = ibufs[b][0]
        rows, sem = rbufs[b]
        return pltpu.make_async_copy(hh_hbm.at[ib.at[0]], rows, sem)

    def process(i, b):
        ib, eb, _ = ibufs[b]
        rows, _g = rbufs[b]

        @plsc.parallel_loop(0, _C, unroll=8)
        def _scale(r):
            w = eb[pl.ds(r, 16)][0]
            for j in range(8):
                rows[r, pl.ds(j * 16, 16)] = rows[r, pl.ds(j * 16, 16)] * w

        pltpu.sync_copy(rows, acc.at[ib.at[1]], add=True)

    z = jnp.zeros((16,), jnp.float32)

    def zrow(i, carry):
        for j in range(8):
            rows0[i, pl.ds(j * 16, 16)] = z
        return carry

    lax.fori_loop(0, _C, zrow, 0)
    for k in range(_NBLK_PT):
        blk = s * _NBLK_PT + k

        @pl.when(blk < _NBLK)
        def _():
            pltpu.sync_copy(rows0, acc.at[pl.ds(blk * _C, _C)])

    for d in idxload(0, 0) + idxload(1, 1):
        d.start()
    plsc.subcore_barrier()
    for d in idxload(0, 0):
        d.wait()
    gather(0, 0).start()

    def pair(k, carry):
        g = k * 2
        for b in range(2):
            i = g + b
            gather(i, b).wait()
            for d in idxload(i + 1, 1 - b):
                d.wait()
            gather(i + 1, 1 - b).start()
            process(i, b)

            @pl.when(i + 2 < _NCHUNK)
            def _():
                for d in idxload(i + 2, b):
                    d.start()
        return carry

    lax.fori_loop(0, (_NCHUNK - 1) // 2, pair, 0)
    last = _NCHUNK - 1
    gather(last, last % 2).wait()
    process(last, last % 2)

    plsc.subcore_barrier()
    for k in range(_NBLK_PT):
        blk = s * _NBLK_PT + k

        @pl.when(blk < _NBLK)
        def _():
            pltpu.sync_copy(acc.at[pl.ds(blk * _C, _C)],
                            out_hbm.at[c, pl.ds(blk * _C, _C)])


_mp_scatter = pl.kernel(
    _mp_scatter_body,
    out_type=jax.ShapeDtypeStruct((_NC, _N, _D), jnp.float32),
    mesh=_sc_mesh,
    scratch_types=[
        pltpu.VMEM((2, _C), jnp.int32),
        pltpu.VMEM((2, _C), jnp.int32),
        pltpu.VMEM((_C + 16,), jnp.float32),
        pltpu.VMEM((_C + 16,), jnp.float32),
        pltpu.VMEM((_C, _D), jnp.float32),
        pltpu.VMEM((_C, _D), jnp.float32),
        pltpu.VMEM_SHARED((_N, _D), jnp.float32),
        pltpu.SemaphoreType.DMA,
        pltpu.SemaphoreType.DMA,
        pltpu.SemaphoreType.DMA,
        pltpu.SemaphoreType.DMA,
    ],
)


def _make_edge_gather(n_edges, c):
    epw = n_edges // _NW
    nchunk = epw // c

    def body(hh_hbm, sd3_hbm, hu_hbm, hv_hbm,
             ib0, ib1, ru0, ru1, rv0, rv1, hh_s,
             is0, is1, wu0, wu1, wv0, wv1):
        cc = lax.axis_index("c")
        s = lax.axis_index("s")
        wid = s * _NC + cc

        ibufs = ((ib0, is0), (ib1, is1))
        rbufs = ((ru0, rv0, wu0, wv0), (ru1, rv1, wu1, wv1))

        def idxload(i, b):
            ib, sem = ibufs[b]
            return pltpu.make_async_copy(sd3_hbm.at[wid, i], ib, sem)

        def writes(i, b):
            ru, rv, wu, wv = rbufs[b]
            base = wid * epw + i * c
            return (pltpu.make_async_copy(ru, hu_hbm.at[pl.ds(base, c)], wu),
                    pltpu.make_async_copy(rv, hv_hbm.at[pl.ds(base, c)], wv))

        idxload(0, 0).start()
        idxload(1, 1).start()
        for k in range(_NBLK_PT):
            blk = s * _NBLK_PT + k

            @pl.when(blk < _NBLK)
            def _():
                pltpu.sync_copy(hh_hbm.at[pl.ds(blk * _C, _C)],
                                hh_s.at[pl.ds(blk * _C, _C)])
        plsc.subcore_barrier()

        def pair(k, carry):
            g = k * 2
            for b in range(2):
                i = g + b
                ib, _ = ibufs[b]
                ru, rv, _wu, _wv = rbufs[b]
                idxload(i, b).wait()

                @pl.when(i >= 2)
                def _():
                    for d in writes(i - 2, b):
                        d.wait()

                pltpu.sync_copy(hh_s.at[ib.at[0]], ru)
                pltpu.sync_copy(hh_s.at[ib.at[1]], rv)

                @pl.when(i + 2 < nchunk)
                def _():
                    idxload(i + 2, b).start()

                for d in writes(i, b):
                    d.start()
            return carry

        lax.fori_loop(0, (nchunk - 1) // 2, pair, 0)

        last = nchunk - 1
        lb = last % 2
        ib, _ = ibufs[lb]
        ru, rv, _wu, _wv = rbufs[lb]
        idxload(last, lb).wait()
        for d in writes(last - 2, lb):
            d.wait()
        pltpu.sync_copy(hh_s.at[ib.at[0]], ru)
        pltpu.sync_copy(hh_s.at[ib.at[1]], rv)
        for d in writes(last, lb):
            d.start()
        for d in writes(last - 1, 1 - lb) + writes(last, lb):
            d.wait()

    return pl.kernel(
        body,
        out_type=(jax.ShapeDtypeStruct((n_edges, _D), jnp.float32),
                  jax.ShapeDtypeStruct((n_edges, _D), jnp.float32)),
        mesh=_sc_mesh,
        scratch_types=[
            pltpu.VMEM((2, c), jnp.int32),
            pltpu.VMEM((2, c), jnp.int32),
            pltpu.VMEM((c, _D), jnp.float32),
            pltpu.VMEM((c, _D), jnp.float32),
            pltpu.VMEM((c, _D), jnp.float32),
            pltpu.VMEM((c, _D), jnp.float32),
            pltpu.VMEM_SHARED((_N, _D), jnp.float32),
            pltpu.SemaphoreType.DMA,
            pltpu.SemaphoreType.DMA,
            pltpu.SemaphoreType.DMA,
            pltpu.SemaphoreType.DMA,
            pltpu.SemaphoreType.DMA,
            pltpu.SemaphoreType.DMA,
        ],
    )


_EG = 2
_EPG = _E // _EG
_CG = 40
_edge_gather_g = _make_edge_gather(_EPG, _CG)


def _mp_combine_body(hh_ref, a0_ref, a1_ref, n_ref, wTl_ref, wTr_ref,
                     b_ref, g_ref, bb_ref, o_ref):
    ah = (a0_ref[...] + a1_ref[...]) * n_ref[...]
    y = (jnp.dot(hh_ref[...], wTl_ref[...], preferred_element_type=jnp.float32)
         + jnp.dot(ah, wTr_ref[...], preferred_element_type=jnp.float32)
         + b_ref[...])
    y = _ln_rows(y, g_ref[...], bb_ref[...])
    o_ref[...] = jnp.maximum(y, 0.0)


def _run_mp_combine(hh, a0, a1, norm, wTl, wTr, b, g, bb):
    return pl.pallas_call(
        _mp_combine_body,
        grid=(_N // _BR,),
        in_specs=[
            pl.BlockSpec((_BR, _D), lambda i: (i, 0)),
            pl.BlockSpec((_BR, _D), lambda i: (i, 0)),
            pl.BlockSpec((_BR, _D), lambda i: (i, 0)),
            pl.BlockSpec((_BR, 1), lambda i: (i, 0)),
            pl.BlockSpec((_D, _D), lambda i: (0, 0)),
            pl.BlockSpec((_D, _D), lambda i: (0, 0)),
            pl.BlockSpec((_D,), lambda i: (0,)),
            pl.BlockSpec((_D,), lambda i: (0,)),
            pl.BlockSpec((_D,), lambda i: (0,)),
        ],
        out_specs=pl.BlockSpec((_BR, _D), lambda i: (i, 0)),
        out_shape=jax.ShapeDtypeStruct((_N, _D), jnp.float32),
    )(hh, a0, a1, norm, wTl, wTr, b, g, bb)


def _edge_mlp_body(hu_ref, hv_ref, ef_ref, w1u_ref, w1v_ref, b1_ref,
                   g_ref, bb_ref, w2a_ref, w2b_ref, b2_ref, o_ref):
    x = (jnp.dot(hu_ref[...], w1u_ref[...], preferred_element_type=jnp.float32)
         + jnp.dot(hv_ref[...], w1v_ref[...], preferred_element_type=jnp.float32)
         + b1_ref[...])
    x = jnp.maximum(_ln_rows(x, g_ref[...], bb_ref[...]), 0.0)
    sc = (jnp.dot(x, w2a_ref[...], preferred_element_type=jnp.float32)
          + jnp.dot(ef_ref[...], w2b_ref[...], preferred_element_type=jnp.float32)
          + b2_ref[...])
    o_ref[...] = sc


def _run_edge_mlp(hu, hv, ef, w1uT, w1vT, b1, g, bb, w2aT, w2bT, b2):
    n_edges = hu.shape[0]
    return pl.pallas_call(
        _edge_mlp_body,
        grid=(n_edges // _BE,),
        in_specs=[
            pl.BlockSpec((_BE, _D), lambda i: (i, 0)),
            pl.BlockSpec((_BE, _D), lambda i: (i, 0)),
            pl.BlockSpec((_BE, 6), lambda i: (i, 0)),
            pl.BlockSpec((_D, 256), lambda i: (0, 0)),
            pl.BlockSpec((_D, 256), lambda i: (0, 0)),
            pl.BlockSpec((256,), lambda i: (0,)),
            pl.BlockSpec((256,), lambda i: (0,)),
            pl.BlockSpec((256,), lambda i: (0,)),
            pl.BlockSpec((256, 5), lambda i: (0, 0)),
            pl.BlockSpec((6, 5), lambda i: (0, 0)),
            pl.BlockSpec((5,), lambda i: (0,)),
        ],
        out_specs=pl.BlockSpec((_BE, 5), lambda i: (i, 0)),
        out_shape=jax.ShapeDtypeStruct((n_edges, 5), jnp.float32),
    )(hu, hv, ef, w1uT, w1vT, b1, g, bb, w2aT, w2bT, b2)


def kernel(h, edge_index, edge_w, norm, edge_feat, proj_w, proj_b, proj_ln_g,
           proj_ln_b, mp_w, mp_b, mp_ln_g, mp_ln_b, W1, b1, ln_g, ln_b, W2, b2):
    src = edge_index[0]
    dst = edge_index[1]
    sd3 = jnp.stack([src.reshape(_NW, _NCHUNK, _C),
                     dst.reshape(_NW, _NCHUNK, _C)], axis=2)
    ew = edge_w[:, 0]

    proj_wT = jnp.swapaxes(proj_w, 1, 2)
    hh = _run_proj(h, proj_wT, proj_b, proj_ln_g, proj_ln_b)

    for l in range(2):
        part = _mp_scatter(hh, sd3, ew)
        hh = _run_mp_combine(hh, part[0], part[1], norm,
                             mp_w[l][:, :128].T, mp_w[l][:, 128:].T,
                             mp_b[l], mp_ln_g[l], mp_ln_b[l])

    sd4 = jnp.stack([src.reshape(_EG, _NW, _EPG // _NW // _CG, _CG),
                     dst.reshape(_EG, _NW, _EPG // _NW // _CG, _CG)], axis=3)
    scores = []
    for g in range(_EG):
        hu, hv = _edge_gather_g(hh, sd4[g])
        scores.append(_run_edge_mlp(
            hu, hv, lax.dynamic_slice_in_dim(edge_feat, g * _EPG, _EPG),
            W1[:, :128].T, W1[:, 128:].T, b1, ln_g, ln_b,
            W2[:, :256].T, W2[:, 256:].T, b2))
    return jnp.concatenate(scores, axis=0)


# speedup vs baseline: 4.8034x; 1.0051x over previous
import jax
import jax.numpy as jnp
from jax import lax
from jax.experimental import pallas as pl
from jax.experimental.pallas import tpu as pltpu
from jax.experimental.pallas import tpu_sc as plsc

_N = 10000
_E = 320000
_D = 128
_NC = 2
_NS = 16
_NW = _NC * _NS
_EPW = _E // _NW
_C = 80
_NCHUNK = _EPW // _C
_NBLK = _N // _C
_NBLK_PT = -(-_NBLK // _NS)
_BR = 1000
_BE = 2560


def _ln_rows(y, g, b):
    m = jnp.mean(y, axis=-1, keepdims=True)
    v = jnp.mean(y * y, axis=-1, keepdims=True) - m * m
    return (y - m) * lax.rsqrt(v + 1e-5) * g + b


def _proj_body(h_ref, wT_ref, b_ref, g_ref, bb_ref, o_ref):
    x = h_ref[...]
    for i in range(2):
        y = jnp.dot(x[:, i * 64:(i + 1) * 64], wT_ref[i],
                    preferred_element_type=jnp.float32) + b_ref[i]
        y = _ln_rows(y, g_ref[i], bb_ref[i])
        o_ref[:, i * 64:(i + 1) * 64] = jnp.maximum(y, 0.0)


def _run_proj(h, proj_wT, proj_b, proj_ln_g, proj_ln_b):
    return pl.pallas_call(
        _proj_body,
        grid=(_N // _BR,),
        in_specs=[
            pl.BlockSpec((_BR, _D), lambda i: (i, 0)),
            pl.BlockSpec((2, 64, 64), lambda i: (0, 0, 0)),
            pl.BlockSpec((2, 64), lambda i: (0, 0)),
            pl.BlockSpec((2, 64), lambda i: (0, 0)),
            pl.BlockSpec((2, 64), lambda i: (0, 0)),
        ],
        out_specs=pl.BlockSpec((_BR, _D), lambda i: (i, 0)),
        out_shape=jax.ShapeDtypeStruct((_N, _D), jnp.float32),
    )(h, proj_wT, proj_b, proj_ln_g, proj_ln_b)


_sc_mesh = plsc.VectorSubcoreMesh(core_axis_name="c", subcore_axis_name="s")


def _mp_scatter_body(hh_hbm, sd3_hbm, ew_hbm, out_hbm,
                     ib0, ib1, eb0, eb1, rows0, rows1, acc,
                     is0, is1, gs0, gs1):
    c = lax.axis_index("c")
    s = lax.axis_index("s")
    wid = s * _NC + c

    ibufs = ((ib0, eb0, is0), (ib1, eb1, is1))
    rbufs = ((rows0, gs0), (rows1, gs1))

    def idxload(i, b):
        ib, eb, sem = ibufs[b]
        base = wid * _EPW + i * _C
        return (pltpu.make_async_copy(sd3_hbm.at[wid, i], ib, sem),
                pltpu.make_async_copy(ew_hbm.at[pl.ds(base, _C)],
                                      eb.at[pl.ds(0, _C)], sem))

    def gather(i, b):
        ib = ibufs[b][0]
        rows, sem = rbufs[b]
        return pltpu.make_async_copy(hh_hbm.at[ib.at[0]], rows, sem)

    def process(i, b):
        ib, eb, _ = ibufs[b]
        rows, _g = rbufs[b]

        @plsc.parallel_loop(0, _C, unroll=8)
        def _scale(r):
            w = eb[pl.ds(r, 16)][0]
            for j in range(8):
                rows[r, pl.ds(j * 16, 16)] = rows[r, pl.ds(j * 16, 16)] * w

        pltpu.sync_copy(rows, acc.at[ib.at[1]], add=True)

    z = jnp.zeros((16,), jnp.float32)

    def zrow(i, carry):
        for j in range(8):
            rows0[i, pl.ds(j * 16, 16)] = z
        return carry

    lax.fori_loop(0, _C, zrow, 0)
    for k in range(_NBLK_PT):
        blk = s * _NBLK_PT + k

        @pl.when(blk < _NBLK)
        def _():
            pltpu.sync_copy(rows0, acc.at[pl.ds(blk * _C, _C)])

    for d in idxload(0, 0) + idxload(1, 1):
        d.start()
    plsc.subcore_barrier()
    for d in idxload(0, 0):
        d.wait()
    gather(0, 0).start()

    def pair(k, carry):
        g = k * 2
        for b in range(2):
            i = g + b
            gather(i, b).wait()
            for d in idxload(i + 1, 1 - b):
                d.wait()
            gather(i + 1, 1 - b).start()
            process(i, b)

            @pl.when(i + 2 < _NCHUNK)
            def _():
                for d in idxload(i + 2, b):
                    d.start()
        return carry

    lax.fori_loop(0, (_NCHUNK - 1) // 2, pair, 0)
    last = _NCHUNK - 1
    gather(last, last % 2).wait()
    process(last, last % 2)

    plsc.subcore_barrier()
    for k in range(_NBLK_PT):
        blk = s * _NBLK_PT + k

        @pl.when(blk < _NBLK)
        def _():
            pltpu.sync_copy(acc.at[pl.ds(blk * _C, _C)],
                            out_hbm.at[c, pl.ds(blk * _C, _C)])


_mp_scatter = pl.kernel(
    _mp_scatter_body,
    out_type=jax.ShapeDtypeStruct((_NC, _N, _D), jnp.float32),
    mesh=_sc_mesh,
    scratch_types=[
        pltpu.VMEM((2, _C), jnp.int32),
        pltpu.VMEM((2, _C), jnp.int32),
        pltpu.VMEM((_C + 16,), jnp.float32),
        pltpu.VMEM((_C + 16,), jnp.float32),
        pltpu.VMEM((_C, _D), jnp.float32),
        pltpu.VMEM((_C, _D), jnp.float32),
        pltpu.VMEM_SHARED((_N, _D), jnp.float32),
        pltpu.SemaphoreType.DMA,
        pltpu.SemaphoreType.DMA,
        pltpu.SemaphoreType.DMA,
        pltpu.SemaphoreType.DMA,
    ],
)


def _make_edge_gather(n_edges, c):
    epw = n_edges // _NW
    nchunk = epw // c

    def body(hh_hbm, sd3_hbm, hu_hbm, hv_hbm,
             ib0, ib1, ru0, ru1, rv0, rv1, hh_s,
             is0, is1, wu0, wu1, wv0, wv1):
        cc = lax.axis_index("c")
        s = lax.axis_index("s")
        wid = s * _NC + cc

        ibufs = ((ib0, is0), (ib1, is1))
        rbufs = ((ru0, rv0, wu0, wv0), (ru1, rv1, wu1, wv1))

        def idxload(i, b):
            ib, sem = ibufs[b]
            return pltpu.make_async_copy(sd3_hbm.at[wid, i], ib, sem)

        def writes(i, b):
            ru, rv, wu, wv = rbufs[b]
            base = wid * epw + i * c
            return (pltpu.make_async_copy(ru, hu_hbm.at[pl.ds(base, c)], wu),
                    pltpu.make_async_copy(rv, hv_hbm.at[pl.ds(base, c)], wv))

        idxload(0, 0).start()
        idxload(1, 1).start()
        for k in range(_NBLK_PT):
            blk = s * _NBLK_PT + k

            @pl.when(blk < _NBLK)
            def _():
                pltpu.sync_copy(hh_hbm.at[pl.ds(blk * _C, _C)],
                                hh_s.at[pl.ds(blk * _C, _C)])
        plsc.subcore_barrier()

        def pair(k, carry):
            g = k * 2
            for b in range(2):
                i = g + b
                ib, _ = ibufs[b]
                ru, rv, _wu, _wv = rbufs[b]
                idxload(i, b).wait()

                @pl.when(i >= 2)
                def _():
                    for d in writes(i - 2, b):
                        d.wait()

                pltpu.sync_copy(hh_s.at[ib.at[0]], ru)
                pltpu.sync_copy(hh_s.at[ib.at[1]], rv)

                @pl.when(i + 2 < nchunk)
                def _():
                    idxload(i + 2, b).start()

                for d in writes(i, b):
                    d.start()
            return carry

        lax.fori_loop(0, (nchunk - 1) // 2, pair, 0)

        last = nchunk - 1
        lb = last % 2
        ib, _ = ibufs[lb]
        ru, rv, _wu, _wv = rbufs[lb]
        idxload(last, lb).wait()
        for d in writes(last - 2, lb):
            d.wait()
        pltpu.sync_copy(hh_s.at[ib.at[0]], ru)
        pltpu.sync_copy(hh_s.at[ib.at[1]], rv)
        for d in writes(last, lb):
            d.start()
        for d in writes(last - 1, 1 - lb) + writes(last, lb):
            d.wait()

    return pl.kernel(
        body,
        out_type=(jax.ShapeDtypeStruct((n_edges, _D), jnp.float32),
                  jax.ShapeDtypeStruct((n_edges, _D), jnp.float32)),
        mesh=_sc_mesh,
        scratch_types=[
            pltpu.VMEM((2, c), jnp.int32),
            pltpu.VMEM((2, c), jnp.int32),
            pltpu.VMEM((c, _D), jnp.float32),
            pltpu.VMEM((c, _D), jnp.float32),
            pltpu.VMEM((c, _D), jnp.float32),
            pltpu.VMEM((c, _D), jnp.float32),
            pltpu.VMEM_SHARED((_N, _D), jnp.float32),
            pltpu.SemaphoreType.DMA,
            pltpu.SemaphoreType.DMA,
            pltpu.SemaphoreType.DMA,
            pltpu.SemaphoreType.DMA,
            pltpu.SemaphoreType.DMA,
            pltpu.SemaphoreType.DMA,
        ],
    )


_EG_SIZES = (41 * _C * _NW, 41 * _C * _NW, 43 * _C * _NW)
_edge_gathers = {n: _make_edge_gather(n, _C) for n in set(_EG_SIZES)}


def _mp_combine_body(hh_ref, a0_ref, a1_ref, n_ref, wTl_ref, wTr_ref,
                     b_ref, g_ref, bb_ref, o_ref):
    ah = (a0_ref[...] + a1_ref[...]) * n_ref[...]
    y = (jnp.dot(hh_ref[...], wTl_ref[...], preferred_element_type=jnp.float32)
         + jnp.dot(ah, wTr_ref[...], preferred_element_type=jnp.float32)
         + b_ref[...])
    y = _ln_rows(y, g_ref[...], bb_ref[...])
    o_ref[...] = jnp.maximum(y, 0.0)


def _run_mp_combine(hh, a0, a1, norm, wTl, wTr, b, g, bb):
    return pl.pallas_call(
        _mp_combine_body,
        grid=(_N // _BR,),
        in_specs=[
            pl.BlockSpec((_BR, _D), lambda i: (i, 0)),
            pl.BlockSpec((_BR, _D), lambda i: (i, 0)),
            pl.BlockSpec((_BR, _D), lambda i: (i, 0)),
            pl.BlockSpec((_BR, 1), lambda i: (i, 0)),
            pl.BlockSpec((_D, _D), lambda i: (0, 0)),
            pl.BlockSpec((_D, _D), lambda i: (0, 0)),
            pl.BlockSpec((_D,), lambda i: (0,)),
            pl.BlockSpec((_D,), lambda i: (0,)),
            pl.BlockSpec((_D,), lambda i: (0,)),
        ],
        out_specs=pl.BlockSpec((_BR, _D), lambda i: (i, 0)),
        out_shape=jax.ShapeDtypeStruct((_N, _D), jnp.float32),
    )(hh, a0, a1, norm, wTl, wTr, b, g, bb)


def _edge_mlp_body(hu_ref, hv_ref, ef_ref, w1u_ref, w1v_ref, b1_ref,
                   g_ref, bb_ref, w2a_ref, w2b_ref, b2_ref, o_ref):
    x = (jnp.dot(hu_ref[...], w1u_ref[...], preferred_element_type=jnp.float32)
         + jnp.dot(hv_ref[...], w1v_ref[...], preferred_element_type=jnp.float32)
         + b1_ref[...])
    x = jnp.maximum(_ln_rows(x, g_ref[...], bb_ref[...]), 0.0)
    sc = (jnp.dot(x, w2a_ref[...], preferred_element_type=jnp.float32)
          + jnp.dot(ef_ref[...], w2b_ref[...], preferred_element_type=jnp.float32)
          + b2_ref[...])
    o_ref[...] = sc


def _run_edge_mlp(hu, hv, ef, w1uT, w1vT, b1, g, bb, w2aT, w2bT, b2):
    n_edges = hu.shape[0]
    return pl.pallas_call(
        _edge_mlp_body,
        grid=(n_edges // _BE,),
        in_specs=[
            pl.BlockSpec((_BE, _D), lambda i: (i, 0)),
            pl.BlockSpec((_BE, _D), lambda i: (i, 0)),
            pl.BlockSpec((_BE, 6), lambda i: (i, 0)),
            pl.BlockSpec((_D, 256), lambda i: (0, 0)),
            pl.BlockSpec((_D, 256), lambda i: (0, 0)),
            pl.BlockSpec((256,), lambda i: (0,)),
            pl.BlockSpec((256,), lambda i: (0,)),
            pl.BlockSpec((256,), lambda i: (0,)),
            pl.BlockSpec((256, 5), lambda i: (0, 0)),
            pl.BlockSpec((6, 5), lambda i: (0, 0)),
            pl.BlockSpec((5,), lambda i: (0,)),
        ],
        out_specs=pl.BlockSpec((_BE, 5), lambda i: (i, 0)),
        out_shape=jax.ShapeDtypeStruct((n_edges, 5), jnp.float32),
    )(hu, hv, ef, w1uT, w1vT, b1, g, bb, w2aT, w2bT, b2)


def kernel(h, edge_index, edge_w, norm, edge_feat, proj_w, proj_b, proj_ln_g,
           proj_ln_b, mp_w, mp_b, mp_ln_g, mp_ln_b, W1, b1, ln_g, ln_b, W2, b2):
    src = edge_index[0]
    dst = edge_index[1]
    sd3 = jnp.stack([src.reshape(_NW, _NCHUNK, _C),
                     dst.reshape(_NW, _NCHUNK, _C)], axis=2)
    ew = edge_w[:, 0]

    proj_wT = jnp.swapaxes(proj_w, 1, 2)
    hh = _run_proj(h, proj_wT, proj_b, proj_ln_g, proj_ln_b)

    for l in range(2):
        part = _mp_scatter(hh, sd3, ew)
        hh = _run_mp_combine(hh, part[0], part[1], norm,
                             mp_w[l][:, :128].T, mp_w[l][:, 128:].T,
                             mp_b[l], mp_ln_g[l], mp_ln_b[l])

    scores = []
    off = 0
    for size in _EG_SIZES:
        nch = size // _NW // _C
        srcg = lax.dynamic_slice_in_dim(src, off, size)
        dstg = lax.dynamic_slice_in_dim(dst, off, size)
        sdg = jnp.stack([srcg.reshape(_NW, nch, _C),
                         dstg.reshape(_NW, nch, _C)], axis=2)
        hu, hv = _edge_gathers[size](hh, sdg)
        scores.append(_run_edge_mlp(
            hu, hv, lax.dynamic_slice_in_dim(edge_feat, off, size),
            W1[:, :128].T, W1[:, 128:].T, b1, ln_g, ln_b,
            W2[:, :256].T, W2[:, 256:].T, b2))
        off += size
    return jnp.concatenate(scores, axis=0)
